# baseline (speedup 1.0000x reference)
import functools

import jax
import jax.numpy as jnp
from jax import lax
from jax.experimental import pallas as pl
from jax.experimental.pallas import tpu as pltpu
from jax.experimental.pallas import tpu_sc as plsc

N1 = 50000
N2 = 50000
E = 100000
DX = 209
DH = 128
DEH = 64
NT = N1 + N2

EP = 131072
NPP = EP // 2


_TC_PARAMS = pltpu.CompilerParams(dimension_semantics=("arbitrary",))


def _w_spec(shape):
    return pl.BlockSpec(shape, lambda i: (0,) * len(shape))


def _encode_kernel(x_ref, wn_ref, bn_ref, wb_ref, h_ref, p_ref):
    h = jnp.maximum(
        jnp.dot(x_ref[...], wn_ref[...], preferred_element_type=jnp.float32)
        + bn_ref[...], 0.0)
    h_ref[...] = h
    p_ref[...] = jnp.dot(h, wb_ref[...], preferred_element_type=jnp.float32)


def _encode(x, wn, bn, wb):
    br = 2000
    return pl.pallas_call(
        _encode_kernel,
        grid=(NT // br,),
        in_specs=[
            pl.BlockSpec((br, DX), lambda i: (i, 0)),
            _w_spec((DX, DH)), _w_spec((1, DH)),
            _w_spec((DH, DEH)),
        ],
        out_specs=[
            pl.BlockSpec((br, DH), lambda i: (i, 0)),
            pl.BlockSpec((br, DEH), lambda i: (i, 0)),
        ],
        out_shape=[
            jax.ShapeDtypeStruct((NT, DH), jnp.float32),
            jax.ShapeDtypeStruct((NT, DEH), jnp.float32),
        ],
        compiler_params=_TC_PARAMS,
    )(x, wn, bn, wb)


def _combine1_kernel(eap_ref, ps_ref, g_ref, webd_ref, bebd_ref,
                     w1a_ref, w1c_ref, b1_ref, out_ref):
    e0 = jnp.maximum(
        jnp.dot(eap_ref[...], webd_ref[...], preferred_element_type=jnp.float32)
        + bebd_ref[...], 0.0)
    p = ps_ref[...]
    pp = jnp.concatenate([p, p], axis=1)
    gd = jnp.dot(g_ref[...], w1c_ref[...], preferred_element_type=jnp.float32)
    e1 = jnp.maximum(
        jnp.dot(e0, w1a_ref[...], preferred_element_type=jnp.float32)
        + pp + gd + b1_ref[...], 0.0)
    rows = pl.program_id(0) * 1024 + lax.broadcasted_iota(jnp.int32, (1024, 1), 0)
    out_ref[...] = jnp.where(rows < N1, e1, 0.0)


def _combine1(eap, p1, g1p, webd, bebd, w1abd, w1cbd, b1bd):
    br = 1024
    return pl.pallas_call(
        _combine1_kernel,
        grid=(NPP // br,),
        in_specs=[
            pl.BlockSpec((br, 12), lambda i: (i, 0)),
            pl.BlockSpec((br, DEH), lambda i: (i, 0)),
            pl.BlockSpec((br, 2 * DH), lambda i: (i, 0)),
            _w_spec((12, 2 * DEH)), _w_spec((1, 2 * DEH)),
            _w_spec((2 * DEH, 2 * DEH)), _w_spec((2 * DH, 2 * DEH)),
            _w_spec((1, 2 * DEH)),
        ],
        out_specs=pl.BlockSpec((br, 2 * DEH), lambda i: (i, 0)),
        out_shape=jax.ShapeDtypeStruct((NPP, 2 * DEH), jnp.float32),
        compiler_params=_TC_PARAMS,
    )(eap, p1, g1p, webd, bebd, w1abd, w1cbd, b1bd)


def _update_src_kernel(h_ref, e_ref, w2_ref, b2_ref, wp_ref, out_ref):
    e = e_ref[...]
    a = e[:, :DEH] + e[:, DEH:]
    hn = jnp.maximum(
        h_ref[...]
        + jnp.dot(a, w2_ref[...], preferred_element_type=jnp.float32)
        + b2_ref[...], 0.0)
    out_ref[...] = jnp.dot(hn, wp_ref[...], preferred_element_type=jnp.float32)


def _update_src(h0, e1p, w2, b2, wp):
    br = 2000
    return pl.pallas_call(
        _update_src_kernel,
        grid=(N1 // br,),
        in_specs=[
            pl.BlockSpec((br, DH), lambda i: (i, 0)),
            pl.BlockSpec((br, 2 * DEH), lambda i: (i, 0)),
            _w_spec((DEH, DH)), _w_spec((1, DH)), _w_spec((DH, DEH)),
        ],
        out_specs=pl.BlockSpec((br, DEH), lambda i: (i, 0)),
        out_shape=jax.ShapeDtypeStruct((N1, DEH), jnp.float32),
        compiler_params=_TC_PARAMS,
    )(h0, e1p, w2, b2, wp)


def _update_dst_kernel(h_ref, a_ref, w2_ref, b2_ref, out_ref):
    out_ref[...] = jnp.maximum(
        h_ref[...]
        + jnp.dot(a_ref[...], w2_ref[...], preferred_element_type=jnp.float32)
        + b2_ref[...], 0.0)


def _update_dst(h0, agg, w2, b2):
    br = 2000
    return pl.pallas_call(
        _update_dst_kernel,
        grid=(N2 // br,),
        in_specs=[
            pl.BlockSpec((br, DH), lambda i: (i + N1 // br, 0)),
            pl.BlockSpec((br, DEH), lambda i: (i, 0)),
            _w_spec((DEH, DH)), _w_spec((1, DH)),
        ],
        out_specs=pl.BlockSpec((br, DH), lambda i: (i, 0)),
        out_shape=jax.ShapeDtypeStruct((N2, DH), jnp.float32),
        compiler_params=_TC_PARAMS,
    )(h0, agg, w2, b2)


def _final_kernel(e1_ref, ps_ref, g_ref, w1a_ref, w1c_ref, b1_ref,
                  wa_ref, ba_ref, wb_ref, bb_ref, wc_ref, bc_ref,
                  wf_ref, bf_ref, out_ref):
    p = ps_ref[...]
    pp = jnp.concatenate([p, p], axis=1)
    gd = jnp.dot(g_ref[...], w1c_ref[...], preferred_element_type=jnp.float32)
    e2 = jnp.maximum(
        jnp.dot(e1_ref[...], w1a_ref[...], preferred_element_type=jnp.float32)
        + pp + gd + b1_ref[...], 0.0)
    z = jnp.maximum(
        jnp.dot(e2, wa_ref[...], preferred_element_type=jnp.float32)
        + ba_ref[...], 0.0)
    z = jnp.maximum(
        jnp.dot(z, wb_ref[...], preferred_element_type=jnp.float32)
        + bb_ref[...], 0.0)
    z = jnp.maximum(
        jnp.dot(z, wc_ref[...], preferred_element_type=jnp.float32)
        + bc_ref[...], 0.0)
    out_ref[...] = (jnp.dot(z, wf_ref[...], preferred_element_type=jnp.float32)
                    + bf_ref[...])


def _final(e1p, p2s, g2p, w1abd, w1cbd, b1bd, cw):
    br = 2000
    return pl.pallas_call(
        _final_kernel,
        grid=(N1 // br,),
        in_specs=[
            pl.BlockSpec((br, 2 * DEH), lambda i: (i, 0)),
            pl.BlockSpec((br, DEH), lambda i: (i, 0)),
            pl.BlockSpec((br, 2 * DH), lambda i: (i, 0)),
            _w_spec((2 * DEH, 2 * DEH)), _w_spec((2 * DH, 2 * DEH)),
            _w_spec((1, 2 * DEH)),
            _w_spec((128, 128)), _w_spec((1, 128)),
            _w_spec((128, 64)), _w_spec((1, 64)),
            _w_spec((64, 32)), _w_spec((1, 32)),
            _w_spec((32, 3)), _w_spec((1, 3)),
        ],
        out_specs=pl.BlockSpec((br, 3), lambda i: (i, 0)),
        out_shape=jax.ShapeDtypeStruct((N1, 3), jnp.float32),
        compiler_params=_TC_PARAMS,
    )(e1p, p2s, g2p, w1abd, w1cbd, b1bd, *cw)


_SC_MESH = dict(core_axis_name="c", subcore_axis_name="s")
_NC, _NS = 2, 16
_NW = _NC * _NS
_GB = EP // _NW
_GCH = _GB // 128
_GG = 4


def _sc_gather(table, idx_flat):
    mesh = plsc.VectorSubcoreMesh(**_SC_MESH)

    @functools.partial(
        pl.kernel, mesh=mesh,
        out_type=jax.ShapeDtypeStruct((EP, DH), jnp.float32),
        scratch_types=[
            pltpu.VMEM((_GB,), jnp.int32),
            pltpu.VMEM((_GG * 128, DH), jnp.float32),
            pltpu.SemaphoreType.DMA,
        ],
    )
    def gather_k(table_hbm, idx_hbm, out_hbm, idx_v, rows_v, sem):
        wid = lax.axis_index("s") * _NC + lax.axis_index("c")
        base = wid * _GB
        pltpu.sync_copy(idx_hbm.at[pl.ds(base, _GB)], idx_v)

        @pl.loop(0, _GCH // _GG)
        def _grp(grp):
            off = grp * _GG
            cps = []
            for j in range(_GG):
                cps.append(pltpu.async_copy(
                    table_hbm.at[idx_v.at[pl.ds((off + j) * 128, 128)]],
                    rows_v.at[pl.ds(j * 128, 128)], sem))
            for cp in cps:
                cp.wait()
            pltpu.sync_copy(
                rows_v,
                out_hbm.at[pl.ds(base + off * 128, _GG * 128)])

    return gather_k(table, idx_flat)


_NWIN = 4
_WROWS = 12544
N2X = _NWIN * _WROWS
_DEAD = _WROWS
_TEDG = EP // _NS
_SCHK = 128
_ITERS = _TEDG // _SCHK
_DROWS = _WROWS // _NS
_SLOT = 1024


def _sc_scatter(vals, widx, zrows):
    mesh = plsc.VectorSubcoreMesh(**_SC_MESH)

    @functools.partial(
        pl.kernel, mesh=mesh,
        out_type=jax.ShapeDtypeStruct((N2X, 2 * DEH), jnp.float32),
        scratch_types=[
            pltpu.VMEM((_SCHK,), jnp.int32),
            pltpu.VMEM((_SCHK, 2 * DEH), jnp.float32),
            pltpu.VMEM_SHARED((_WROWS + 8, 2 * DEH), jnp.float32),
        ],
    )
    def scatter_k(v_hbm, widx_hbm, z_hbm, out_hbm, idx_v, buf_v, acc_s):
        c = lax.axis_index("c")
        s = lax.axis_index("s")
        for t in range(2):
            w = c * 2 + t
            pltpu.sync_copy(z_hbm, acc_s.at[pl.ds(s * _DROWS, _DROWS)])
            plsc.subcore_barrier()

            @pl.loop(0, _ITERS)
            def _grp(g):
                slot = (w * _NS + s) * _ITERS + g
                pltpu.sync_copy(
                    widx_hbm.at[pl.ds(slot * _SLOT, _SCHK)], idx_v)
                pltpu.sync_copy(
                    v_hbm.at[pl.ds(s * _TEDG + g * _SCHK, _SCHK)], buf_v)
                pltpu.sync_copy(buf_v, acc_s.at[idx_v], add=True)

            plsc.subcore_barrier()
            pltpu.sync_copy(
                acc_s.at[pl.ds(s * _DROWS, _DROWS)],
                out_hbm.at[pl.ds(w * _WROWS + s * _DROWS, _DROWS)])

    return scatter_k(vals, widx, zrows)


def _blockdiag2(w):
    k, n = w.shape
    z = jnp.zeros((2 * k, 2 * n), w.dtype)
    return z.at[:k, :n].set(w).at[k:, n:].set(w)


def _row(v):
    return v.reshape(1, -1)


def kernel(x, edge_attr, params, edge_index):
    p = params
    st1, st2 = p["steps"][0], p["steps"][1]
    w1a1, w1b1, w1c1 = st1["W1"][:DEH], st1["W1"][DEH:DEH + DH], st1["W1"][DEH + DH:]
    w1a2, w1b2, w1c2 = st2["W1"][:DEH], st2["W1"][DEH:DEH + DH], st2["W1"][DEH + DH:]
    webd = _blockdiag2(p["We"])
    bebd = _row(jnp.concatenate([p["be"], p["be"]]))
    w1abd1 = _blockdiag2(w1a1)
    w1cbd1 = _blockdiag2(w1c1)
    b1bd1 = _row(jnp.concatenate([st1["b1"], st1["b1"]]))
    w1abd2 = _blockdiag2(w1a2)
    w1cbd2 = _blockdiag2(w1c2)
    b1bd2 = _row(jnp.concatenate([st2["b1"], st2["b1"]]))

    hid = p["cls"]["hidden"]
    cw = []
    for layer in hid:
        cw.append(layer["W"] * layer["gamma"][None, :])
        cw.append(_row(layer["b"] * layer["gamma"] + layer["beta"]))
    cw.append(p["cls"]["Wf"])
    cw.append(_row(p["cls"]["bf"]))

    dst = edge_index[1].astype(jnp.int32)
    gidx1 = jnp.zeros((EP,), jnp.int32).at[:E].set(dst)
    gidx2 = jnp.zeros((EP,), jnp.int32).at[:E].set(dst - N1)
    dstrow = jnp.full((EP,), -1, jnp.int32).at[:E].set(dst - N1)
    wins = []
    for w in range(_NWIN):
        local = dstrow - w * _WROWS
        wins.append(jnp.where((local >= 0) & (local < _WROWS), local, _DEAD))
    widx = jnp.pad(
        jnp.stack(wins).reshape(_NWIN, _NS, _ITERS, _SCHK),
        ((0, 0), (0, 0), (0, 0), (0, _SLOT - _SCHK))).reshape(-1)
    eap = jnp.zeros((NPP, 12), jnp.float32).at[:N1].set(
        edge_attr.reshape(N1, 12))
    zrows = jnp.zeros((_DROWS, 2 * DEH), jnp.float32)

    h0, p1 = _encode(x, p["Wn"], _row(p["bn"]), w1b1)
    g1 = _sc_gather(h0, gidx1)
    e1p = _combine1(eap, p1, g1.reshape(NPP, 2 * DH),
                    webd, bebd, w1abd1, w1cbd1, b1bd1)
    e1f = e1p.reshape(EP, DEH)
    e1fw = jnp.pad(e1f, ((0, 0), (0, DEH)))
    agg = _sc_scatter(e1fw, widx, zrows)
    p2s = _update_src(h0, e1p, st1["W2"], _row(st1["b2"]), w1b2)
    h1d = _update_dst(h0, agg[:N2, :DEH], st1["W2"], _row(st1["b2"]))
    g2 = _sc_gather(h1d, gidx2)
    return _final(e1p, p2s, g2.reshape(NPP, 2 * DH), w1abd2, w1cbd2, b1bd2, cw)


# speedup vs baseline: 1.0379x; 1.0379x over previous
import functools

import jax
import jax.numpy as jnp
from jax import lax
from jax.experimental import pallas as pl
from jax.experimental.pallas import tpu as pltpu
from jax.experimental.pallas import tpu_sc as plsc

N1 = 50000
N2 = 50000
E = 100000
DX = 209
DH = 128
DEH = 64
NT = N1 + N2

EP = 131072
NPP = EP // 2


_TC_PARAMS = pltpu.CompilerParams(dimension_semantics=("arbitrary",))


def _w_spec(shape):
    return pl.BlockSpec(shape, lambda i: (0,) * len(shape))


def _encode_kernel(x_ref, wn_ref, bn_ref, wb_ref, h_ref, p_ref):
    h = jnp.maximum(
        jnp.dot(x_ref[...], wn_ref[...], preferred_element_type=jnp.float32)
        + bn_ref[...], 0.0)
    h_ref[...] = h
    p_ref[...] = jnp.dot(h, wb_ref[...], preferred_element_type=jnp.float32)


def _encode(x, wn, bn, wb):
    br = 2000
    return pl.pallas_call(
        _encode_kernel,
        grid=(NT // br,),
        in_specs=[
            pl.BlockSpec((br, DX), lambda i: (i, 0)),
            _w_spec((DX, DH)), _w_spec((1, DH)),
            _w_spec((DH, DEH)),
        ],
        out_specs=[
            pl.BlockSpec((br, DH), lambda i: (i, 0)),
            pl.BlockSpec((br, DEH), lambda i: (i, 0)),
        ],
        out_shape=[
            jax.ShapeDtypeStruct((NT, DH), jnp.float32),
            jax.ShapeDtypeStruct((NT, DEH), jnp.float32),
        ],
        compiler_params=_TC_PARAMS,
    )(x, wn, bn, wb)


def _combine1_kernel(eap_ref, ps_ref, g_ref, webd_ref, bebd_ref,
                     w1a_ref, w1c_ref, b1_ref, out_ref):
    e0 = jnp.maximum(
        jnp.dot(eap_ref[...], webd_ref[...], preferred_element_type=jnp.float32)
        + bebd_ref[...], 0.0)
    p = ps_ref[...]
    pp = jnp.concatenate([p, p], axis=1)
    gd = jnp.dot(g_ref[...], w1c_ref[...], preferred_element_type=jnp.float32)
    e1 = jnp.maximum(
        jnp.dot(e0, w1a_ref[...], preferred_element_type=jnp.float32)
        + pp + gd + b1_ref[...], 0.0)
    rows = pl.program_id(0) * 1024 + lax.broadcasted_iota(jnp.int32, (1024, 1), 0)
    out_ref[...] = jnp.where(rows < N1, e1, 0.0)


def _combine1(eap, p1, g1p, webd, bebd, w1abd, w1cbd, b1bd):
    br = 1024
    return pl.pallas_call(
        _combine1_kernel,
        grid=(NPP // br,),
        in_specs=[
            pl.BlockSpec((br, 12), lambda i: (i, 0)),
            pl.BlockSpec((br, DEH), lambda i: (i, 0)),
            pl.BlockSpec((br, 2 * DH), lambda i: (i, 0)),
            _w_spec((12, 2 * DEH)), _w_spec((1, 2 * DEH)),
            _w_spec((2 * DEH, 2 * DEH)), _w_spec((2 * DH, 2 * DEH)),
            _w_spec((1, 2 * DEH)),
        ],
        out_specs=pl.BlockSpec((br, 2 * DEH), lambda i: (i, 0)),
        out_shape=jax.ShapeDtypeStruct((NPP, 2 * DEH), jnp.float32),
        compiler_params=_TC_PARAMS,
    )(eap, p1, g1p, webd, bebd, w1abd, w1cbd, b1bd)


def _update_src_kernel(h_ref, e_ref, w2_ref, b2_ref, wp_ref, out_ref):
    e = e_ref[...]
    a = e[:, :DEH] + e[:, DEH:]
    hn = jnp.maximum(
        h_ref[...]
        + jnp.dot(a, w2_ref[...], preferred_element_type=jnp.float32)
        + b2_ref[...], 0.0)
    out_ref[...] = jnp.dot(hn, wp_ref[...], preferred_element_type=jnp.float32)


def _update_src(h0, e1p, w2, b2, wp):
    br = 2000
    return pl.pallas_call(
        _update_src_kernel,
        grid=(N1 // br,),
        in_specs=[
            pl.BlockSpec((br, DH), lambda i: (i, 0)),
            pl.BlockSpec((br, 2 * DEH), lambda i: (i, 0)),
            _w_spec((DEH, DH)), _w_spec((1, DH)), _w_spec((DH, DEH)),
        ],
        out_specs=pl.BlockSpec((br, DEH), lambda i: (i, 0)),
        out_shape=jax.ShapeDtypeStruct((N1, DEH), jnp.float32),
        compiler_params=_TC_PARAMS,
    )(h0, e1p, w2, b2, wp)


def _update_dst_kernel(h_ref, a_ref, w2_ref, b2_ref, out_ref):
    out_ref[...] = jnp.maximum(
        h_ref[...]
        + jnp.dot(a_ref[...], w2_ref[...], preferred_element_type=jnp.float32)
        + b2_ref[...], 0.0)


def _update_dst(h0, agg, w2, b2):
    br = 2000
    return pl.pallas_call(
        _update_dst_kernel,
        grid=(N2 // br,),
        in_specs=[
            pl.BlockSpec((br, DH), lambda i: (i + N1 // br, 0)),
            pl.BlockSpec((br, DEH), lambda i: (i, 0)),
            _w_spec((DEH, DH)), _w_spec((1, DH)),
        ],
        out_specs=pl.BlockSpec((br, DH), lambda i: (i, 0)),
        out_shape=jax.ShapeDtypeStruct((N2, DH), jnp.float32),
        compiler_params=_TC_PARAMS,
    )(h0, agg, w2, b2)


def _final_kernel(e1_ref, ps_ref, g_ref, w1a_ref, w1c_ref, b1_ref,
                  wa_ref, ba_ref, wb_ref, bb_ref, wc_ref, bc_ref,
                  wf_ref, bf_ref, out_ref):
    p = ps_ref[...]
    pp = jnp.concatenate([p, p], axis=1)
    gd = jnp.dot(g_ref[...], w1c_ref[...], preferred_element_type=jnp.float32)
    e2 = jnp.maximum(
        jnp.dot(e1_ref[...], w1a_ref[...], preferred_element_type=jnp.float32)
        + pp + gd + b1_ref[...], 0.0)
    z = jnp.maximum(
        jnp.dot(e2, wa_ref[...], preferred_element_type=jnp.float32)
        + ba_ref[...], 0.0)
    z = jnp.maximum(
        jnp.dot(z, wb_ref[...], preferred_element_type=jnp.float32)
        + bb_ref[...], 0.0)
    z = jnp.maximum(
        jnp.dot(z, wc_ref[...], preferred_element_type=jnp.float32)
        + bc_ref[...], 0.0)
    out_ref[...] = (jnp.dot(z, wf_ref[...], preferred_element_type=jnp.float32)
                    + bf_ref[...])


def _final(e1p, p2s, g2p, w1abd, w1cbd, b1bd, cw):
    br = 2000
    return pl.pallas_call(
        _final_kernel,
        grid=(N1 // br,),
        in_specs=[
            pl.BlockSpec((br, 2 * DEH), lambda i: (i, 0)),
            pl.BlockSpec((br, DEH), lambda i: (i, 0)),
            pl.BlockSpec((br, 2 * DH), lambda i: (i, 0)),
            _w_spec((2 * DEH, 2 * DEH)), _w_spec((2 * DH, 2 * DEH)),
            _w_spec((1, 2 * DEH)),
            _w_spec((128, 128)), _w_spec((1, 128)),
            _w_spec((128, 64)), _w_spec((1, 64)),
            _w_spec((64, 32)), _w_spec((1, 32)),
            _w_spec((32, 3)), _w_spec((1, 3)),
        ],
        out_specs=pl.BlockSpec((br, 3), lambda i: (i, 0)),
        out_shape=jax.ShapeDtypeStruct((N1, 3), jnp.float32),
        compiler_params=_TC_PARAMS,
    )(e1p, p2s, g2p, w1abd, w1cbd, b1bd, *cw)


_SC_MESH = dict(core_axis_name="c", subcore_axis_name="s")
_NC, _NS = 2, 16
_NW = _NC * _NS
_GB = EP // _NW
_GCH = _GB // 128
_GG = 4


def _sc_gather(table, idx_flat):
    mesh = plsc.VectorSubcoreMesh(**_SC_MESH)

    @functools.partial(
        pl.kernel, mesh=mesh,
        out_type=jax.ShapeDtypeStruct((EP, DH), jnp.float32),
        scratch_types=[
            pltpu.VMEM((_GB,), jnp.int32),
            pltpu.VMEM((_GG * 128, DH), jnp.float32),
            pltpu.SemaphoreType.DMA,
        ],
    )
    def gather_k(table_hbm, idx_hbm, out_hbm, idx_v, rows_v, sem):
        wid = lax.axis_index("s") * _NC + lax.axis_index("c")
        base = wid * _GB
        pltpu.sync_copy(idx_hbm.at[pl.ds(base, _GB)], idx_v)

        @pl.loop(0, _GCH // _GG)
        def _grp(grp):
            off = grp * _GG
            cps = []
            for j in range(_GG):
                cps.append(pltpu.async_copy(
                    table_hbm.at[idx_v.at[pl.ds((off + j) * 128, 128)]],
                    rows_v.at[pl.ds(j * 128, 128)], sem))
            for cp in cps:
                cp.wait()
            pltpu.sync_copy(
                rows_v,
                out_hbm.at[pl.ds(base + off * 128, _GG * 128)])

    return gather_k(table, idx_flat)


_NWIN = 2
_WROWS = 12544
_WDST = 2 * _WROWS
N2X = _NWIN * _WDST
_DEAD = _WROWS
_TEDG = EP // _NS
_SCHK = 128
_ITERS = _TEDG // _SCHK
_DROWS = _WROWS // _NS
_SLOT = 1024


def _sc_scatter(vals, widx, zrows):
    mesh = plsc.VectorSubcoreMesh(**_SC_MESH)

    @functools.partial(
        pl.kernel, mesh=mesh,
        out_type=jax.ShapeDtypeStruct((_NWIN * _WROWS, 2 * DEH), jnp.float32),
        scratch_types=[
            pltpu.VMEM((_SCHK,), jnp.int32),
            pltpu.VMEM((_SCHK, 2 * DEH), jnp.float32),
            pltpu.VMEM_SHARED((_WROWS + 8, 2 * DEH), jnp.float32),
        ],
    )
    def scatter_k(v_hbm, widx_hbm, z_hbm, out_hbm, idx_v, buf_v, acc_s):
        c = lax.axis_index("c")
        s = lax.axis_index("s")
        pltpu.sync_copy(z_hbm, acc_s.at[pl.ds(s * _DROWS, _DROWS)])
        plsc.subcore_barrier()

        @pl.loop(0, _ITERS)
        def _grp(g):
            slot = (c * _NS + s) * _ITERS + g
            pltpu.sync_copy(
                widx_hbm.at[pl.ds(slot * _SLOT, _SCHK)], idx_v)
            pltpu.sync_copy(
                v_hbm.at[pl.ds(s * _TEDG + g * _SCHK, _SCHK)], buf_v)
            pltpu.sync_copy(buf_v, acc_s.at[idx_v], add=True)

        plsc.subcore_barrier()
        pltpu.sync_copy(
            acc_s.at[pl.ds(s * _DROWS, _DROWS)],
            out_hbm.at[pl.ds(c * _WROWS + s * _DROWS, _DROWS)])

    return scatter_k(vals, widx, zrows)


def _blockdiag2(w):
    k, n = w.shape
    z = jnp.zeros((2 * k, 2 * n), w.dtype)
    return z.at[:k, :n].set(w).at[k:, n:].set(w)


def _row(v):
    return v.reshape(1, -1)


def kernel(x, edge_attr, params, edge_index):
    p = params
    st1, st2 = p["steps"][0], p["steps"][1]
    w1a1, w1b1, w1c1 = st1["W1"][:DEH], st1["W1"][DEH:DEH + DH], st1["W1"][DEH + DH:]
    w1a2, w1b2, w1c2 = st2["W1"][:DEH], st2["W1"][DEH:DEH + DH], st2["W1"][DEH + DH:]
    webd = _blockdiag2(p["We"])
    bebd = _row(jnp.concatenate([p["be"], p["be"]]))
    w1abd1 = _blockdiag2(w1a1)
    w1cbd1 = _blockdiag2(w1c1)
    b1bd1 = _row(jnp.concatenate([st1["b1"], st1["b1"]]))
    w1abd2 = _blockdiag2(w1a2)
    w1cbd2 = _blockdiag2(w1c2)
    b1bd2 = _row(jnp.concatenate([st2["b1"], st2["b1"]]))

    hid = p["cls"]["hidden"]
    cw = []
    for layer in hid:
        cw.append(layer["W"] * layer["gamma"][None, :])
        cw.append(_row(layer["b"] * layer["gamma"] + layer["beta"]))
    cw.append(p["cls"]["Wf"])
    cw.append(_row(p["cls"]["bf"]))

    dst = edge_index[1].astype(jnp.int32)
    gidx1 = jnp.zeros((EP,), jnp.int32).at[:E].set(dst)
    gidx2 = jnp.zeros((EP,), jnp.int32).at[:E].set(dst - N1)
    dstrow = jnp.full((EP,), -1, jnp.int32).at[:E].set(dst - N1)
    wins = []
    for w in range(_NWIN):
        local = dstrow - w * _WDST
        wins.append(jnp.where(
            (local >= 0) & (local < _WDST), local >> 1, _DEAD))
    widx = jnp.pad(
        jnp.stack(wins).reshape(_NWIN, _NS, _ITERS, _SCHK),
        ((0, 0), (0, 0), (0, 0), (0, _SLOT - _SCHK))).reshape(-1)
    even = (dstrow & 1) == 0
    eap = jnp.zeros((NPP, 12), jnp.float32).at[:N1].set(
        edge_attr.reshape(N1, 12))
    zrows = jnp.zeros((_DROWS, 2 * DEH), jnp.float32)

    h0, p1 = _encode(x, p["Wn"], _row(p["bn"]), w1b1)
    g1 = _sc_gather(h0, gidx1)
    e1p = _combine1(eap, p1, g1.reshape(NPP, 2 * DH),
                    webd, bebd, w1abd1, w1cbd1, b1bd1)
    e1f = e1p.reshape(EP, DEH)
    z64 = jnp.zeros_like(e1f)
    e1fw = jnp.where(even[:, None],
                     jnp.concatenate([e1f, z64], axis=1),
                     jnp.concatenate([z64, e1f], axis=1))
    agg = _sc_scatter(e1fw, widx, zrows).reshape(N2X, DEH)
    p2s = _update_src(h0, e1p, st1["W2"], _row(st1["b2"]), w1b2)
    h1d = _update_dst(h0, agg[:N2], st1["W2"], _row(st1["b2"]))
    g2 = _sc_gather(h1d, gidx2)
    return _final(e1p, p2s, g2.reshape(NPP, 2 * DH), w1abd2, w1cbd2, b1bd2, cw)
